# baseline (speedup 1.0000x reference)
import functools

import jax
import jax.numpy as jnp
from jax import lax
from jax.experimental import pallas as pl
from jax.experimental.pallas import tpu as pltpu
from jax.experimental.pallas import tpu_sc as plsc

EMBEDDING_SIZE = 64
BATCH = 16384
HIST_LEN = 50

_info = plsc.get_sparse_core_info()
_NC, _NS = _info.num_cores, _info.num_subcores
_NW = _NC * _NS
_ROWS_W = BATCH // _NW

_G = 8
_NG = _ROWS_W // _G


def _make_gather():
    mesh = plsc.VectorSubcoreMesh(core_axis_name="c", subcore_axis_name="s")

    @functools.partial(
        pl.kernel,
        mesh=mesh,
        compiler_params=pltpu.CompilerParams(use_tc_tiling_on_sc=False),
        out_type=jax.ShapeDtypeStruct(
            (BATCH, HIST_LEN, EMBEDDING_SIZE), jnp.float32
        ),
        scratch_types=[
            pltpu.VMEM((_ROWS_W, HIST_LEN), jnp.int32),
            pltpu.VMEM((_G, HIST_LEN, EMBEDDING_SIZE), jnp.float32),
            pltpu.VMEM((_G, HIST_LEN, EMBEDDING_SIZE), jnp.float32),
            pltpu.SemaphoreType.DMA,
            pltpu.SemaphoreType.DMA,
        ],
    )
    def gather_kernel(idx_hbm, table_hbm, out_hbm, idx_v, rows0, rows1,
                      gsem, ssem):
        wid = lax.axis_index("s") * _NC + lax.axis_index("c")
        row0 = wid * _ROWS_W
        pltpu.sync_copy(idx_hbm.at[pl.ds(row0, _ROWS_W)], idx_v)

        rows = (rows0, rows1)

        def do_group(g, set_i, drain_prev):
            buf = rows[set_i]
            descs = [
                pltpu.async_copy(
                    table_hbm.at[idx_v.at[g * _G + r]],
                    buf.at[r],
                    gsem,
                )
                for r in range(_G)
            ]
            for d in descs:
                d.wait()
            if drain_prev:
                pltpu.make_async_copy(
                    rows[1 - set_i], out_hbm.at[pl.ds(0, _G)], ssem
                ).wait()
            pltpu.async_copy(
                buf, out_hbm.at[pl.ds(row0 + g * _G, _G)], ssem
            )

        do_group(0, 0, False)
        do_group(1, 1, True)

        def body(t, carry):
            do_group(2 * t + 2, 0, True)
            do_group(2 * t + 3, 1, True)
            return carry

        lax.fori_loop(0, (_NG - 2) // 2, body, 0)
        pltpu.make_async_copy(
            rows1, out_hbm.at[pl.ds(0, _G)], ssem
        ).wait()

    return gather_kernel


_gather = _make_gather()


def kernel(indices_words, table):
    return _gather(indices_words.astype(jnp.int32), table)
